# baseline (speedup 1.0000x reference)
import functools

import jax
import jax.numpy as jnp
from jax import lax
from jax.experimental import pallas as pl
from jax.experimental.pallas import tpu as pltpu
from jax.experimental.pallas import tpu_sc as plsc

_E = 8
_K = 2
_D = 2048
_INTER = 768
_T = 2048
_BT = 128
_NB = _T * _K // _BT + _E
_NPAD = _NB * _BT
_RB = 256
_CT = 16
_CW = 8


def _router_kernel(x_ref, gw_ref, i1_ref, i2_ref, w1_ref, w2_ref):
    x = x_ref[...]
    logits = lax.dot_general(
        x, gw_ref[...], (((1,), (1,)), ((), ())),
        preferred_element_type=jnp.float32,
        precision=lax.Precision.DEFAULT)
    lane = lax.broadcasted_iota(jnp.int32, logits.shape, 1)
    v1 = jnp.max(logits, axis=-1, keepdims=True)
    i1 = jnp.min(jnp.where(logits >= v1, lane, _E), axis=-1, keepdims=True)
    lm = jnp.where(lane == i1, -jnp.inf, logits)
    v2 = jnp.max(lm, axis=-1, keepdims=True)
    i2 = jnp.min(jnp.where(lm >= v2, lane, _E), axis=-1, keepdims=True)
    r = jnp.exp(v2 - v1)
    i1_ref[...] = i1
    i2_ref[...] = i2
    w1_ref[...] = 1.0 / (1.0 + r)
    w2_ref[...] = r / (1.0 + r)


def _router(x2d, gate_w):
    return pl.pallas_call(
        _router_kernel,
        grid=(_T // _RB,),
        in_specs=[
            pl.BlockSpec((_RB, _D), lambda i: (i, 0)),
            pl.BlockSpec((_E, _D), lambda i: (0, 0)),
        ],
        out_specs=[
            pl.BlockSpec((_RB, 1), lambda i: (i, 0)),
            pl.BlockSpec((_RB, 1), lambda i: (i, 0)),
            pl.BlockSpec((_RB, 1), lambda i: (i, 0)),
            pl.BlockSpec((_RB, 1), lambda i: (i, 0)),
        ],
        out_shape=[
            jax.ShapeDtypeStruct((_T, 1), jnp.int32),
            jax.ShapeDtypeStruct((_T, 1), jnp.int32),
            jax.ShapeDtypeStruct((_T, 1), jnp.float32),
            jax.ShapeDtypeStruct((_T, 1), jnp.float32),
        ],
    )(x2d, gate_w)


def _routing_metadata(i1, i2, w1, w2):
    ef = jnp.concatenate([i1, i2], axis=1).reshape(-1)
    onehot = (ef[:, None] == jnp.arange(_E, dtype=jnp.int32)[None, :])
    csum = jnp.cumsum(onehot.astype(jnp.int32), axis=0)
    counts = csum[-1]
    rank = jnp.take_along_axis(csum, ef[:, None], axis=1)[:, 0] - 1
    padded = ((counts + _BT - 1) // _BT) * _BT
    bounds = jnp.cumsum(padded)
    pstart = bounds - padded
    pos = (pstart[ef] + rank).astype(jnp.int32)
    nused = (jnp.sum(padded) // _BT).astype(jnp.int32)
    b = jnp.arange(_NB, dtype=jnp.int32)
    step_block = jnp.where(b < nused, b, nused - 1)
    step_expert = jnp.searchsorted(bounds, step_block * _BT,
                                   side='right').astype(jnp.int32)
    pp = pos.reshape(_T, _K)
    return step_block, step_expert, pp[:, 0], pp[:, 1]


_NW = 32
_RPW = _NPAD // _NW
_TPW = _T // _NW


def _sc_dispatch(x2d, p0, p1):
    nch = _TPW // _CT
    i0 = p0.reshape(_NW, nch, _CT)
    i1 = p1.reshape(_NW, nch, _CT)
    mesh = plsc.VectorSubcoreMesh(core_axis_name="core",
                                  subcore_axis_name="subcore")

    @functools.partial(
        pl.kernel,
        out_type=jax.ShapeDtypeStruct((_NPAD, _D), jnp.float32),
        mesh=mesh,
        scratch_types=[
            pltpu.VMEM((nch, _CT), jnp.int32),
            pltpu.VMEM((nch, _CT), jnp.int32),
            pltpu.VMEM((_CT, _D), jnp.float32),
            pltpu.VMEM((_CT, _D), jnp.float32),
            pltpu.SemaphoreType.DMA,
            pltpu.SemaphoreType.DMA,
        ])
    def k(x_hbm, i0_hbm, i1_hbm, o_hbm, idx0_v, idx1_v, b0, b1, semR, semW):
        wid = lax.axis_index("subcore") * 2 + lax.axis_index("core")
        base = wid * _TPW
        pltpu.sync_copy(i0_hbm.at[wid], idx0_v)
        pltpu.sync_copy(i1_hbm.at[wid], idx1_v)

        def rd(c, buf):
            pltpu.async_copy(x_hbm.at[pl.ds(base + c * _CT, _CT)],
                             buf, semR)

        def drain_rd(buf):
            pltpu.make_async_copy(x_hbm.at[pl.ds(base, _CT)],
                                  buf, semR).wait()

        def scat(c, buf):
            pltpu.async_copy(buf, o_hbm.at[idx0_v.at[c]], semW)
            pltpu.async_copy(buf, o_hbm.at[idx1_v.at[c]], semW)
            pltpu.make_async_copy(buf, o_hbm.at[idx0_v.at[c]], semW).wait()
            pltpu.make_async_copy(buf, o_hbm.at[idx0_v.at[c]], semW).wait()

        rd(0, b0)

        @pl.loop(0, nch, step=2)
        def _(c):
            rd(c + 1, b1)
            drain_rd(b0)
            scat(c, b0)

            @pl.when(c + 2 < nch)
            def _():
                rd(c + 2, b0)

            drain_rd(b1)
            scat(c + 1, b1)

    return k(x2d, i0, i1)


def _mlp_kernel(sb_ref, se_ref, x_ref, g_ref, u_ref, d_ref, y_ref):
    i = pl.program_id(0)
    prev = sb_ref[jnp.maximum(i - 1, 0)]
    active = jnp.logical_or(i == 0, sb_ref[i] != prev)

    @pl.when(active)
    def _():
        x = x_ref[...]
        hg = lax.dot_general(x, g_ref[0], (((1,), (1,)), ((), ())),
                             preferred_element_type=jnp.float32,
                             precision=lax.Precision.DEFAULT)
        hu = lax.dot_general(x, u_ref[0], (((1,), (1,)), ((), ())),
                             preferred_element_type=jnp.float32,
                             precision=lax.Precision.DEFAULT)
        h = hg * jax.nn.sigmoid(hg) * hu
        y = lax.dot_general(h, d_ref[0], (((1,), (1,)), ((), ())),
                            preferred_element_type=jnp.float32,
                            precision=lax.Precision.DEFAULT)
        y_ref[...] = y


def _mlp(x_sorted, gw, uw, dw, step_block, step_expert):
    grid_spec = pltpu.PrefetchScalarGridSpec(
        num_scalar_prefetch=2,
        grid=(_NB,),
        in_specs=[
            pl.BlockSpec((_BT, _D), lambda i, sb, se: (sb[i], 0)),
            pl.BlockSpec((1, _INTER, _D), lambda i, sb, se: (se[i], 0, 0)),
            pl.BlockSpec((1, _INTER, _D), lambda i, sb, se: (se[i], 0, 0)),
            pl.BlockSpec((1, _D, _INTER), lambda i, sb, se: (se[i], 0, 0)),
        ],
        out_specs=pl.BlockSpec((_BT, _D), lambda i, sb, se: (sb[i], 0)),
    )
    return pl.pallas_call(
        _mlp_kernel,
        grid_spec=grid_spec,
        out_shape=jax.ShapeDtypeStruct((_NPAD, _D), jnp.float32),
    )(step_block, step_expert, x_sorted, gw, uw, dw)


def _sc_combine(y_sorted, p0, p1, w0, w1):
    mesh = plsc.VectorSubcoreMesh(core_axis_name="core",
                                  subcore_axis_name="subcore")

    nch = _TPW // _CW

    @functools.partial(
        pl.kernel,
        out_type=jax.ShapeDtypeStruct((_T, _D), jnp.float32),
        mesh=mesh,
        scratch_types=[
            pltpu.VMEM((_TPW,), jnp.int32),
            pltpu.VMEM((_TPW,), jnp.int32),
            pltpu.VMEM((_TPW, 16), jnp.float32),
            pltpu.VMEM((_TPW, 16), jnp.float32),
            pltpu.VMEM((_CW, _D), jnp.float32),
            pltpu.VMEM((_CW, _D), jnp.float32),
            pltpu.VMEM((_CW, _D), jnp.float32),
            pltpu.VMEM((_CW, _D), jnp.float32),
            pltpu.SemaphoreType.DMA,
            pltpu.SemaphoreType.DMA,
        ])
    def k(y_hbm, i0_hbm, i1_hbm, w0_hbm, w1_hbm, o_hbm, idx0_v, idx1_v,
          w0_v, w1_v, a0, a1, c0, c1, semA, semB):
        wid = lax.axis_index("subcore") * 2 + lax.axis_index("core")
        base = wid * _TPW
        pltpu.sync_copy(i0_hbm.at[pl.ds(base, _TPW)], idx0_v)
        pltpu.sync_copy(i1_hbm.at[pl.ds(base, _TPW)], idx1_v)
        pltpu.sync_copy(w0_hbm.at[pl.ds(base, _TPW)], w0_v)
        pltpu.sync_copy(w1_hbm.at[pl.ds(base, _TPW)], w1_v)

        def start2(ch, u0, u1, sem):
            s = ch * _CW
            pltpu.async_copy(y_hbm.at[idx0_v.at[pl.ds(s, _CW)]], u0, sem)
            pltpu.async_copy(y_hbm.at[idx1_v.at[pl.ds(s, _CW)]], u1, sem)

        def drain2(u0, u1, sem):
            pltpu.make_async_copy(y_hbm.at[idx0_v.at[pl.ds(0, _CW)]],
                                  u0, sem).wait()
            pltpu.make_async_copy(y_hbm.at[idx0_v.at[pl.ds(0, _CW)]],
                                  u1, sem).wait()

        def addput(ch, u0, u1):
            @pl.loop(0, _CW)
            def _(r):
                ws0 = w0_v[ch * _CW + r]
                ws1 = w1_v[ch * _CW + r]

                @pl.loop(0, _D, step=128)
                def _(cc):
                    for j in range(8):
                        sl = (pl.ds(r, 1), pl.ds(cc + j * 16, 16))
                        u0.at[sl[0], sl[1]][...] = (
                            u0.at[sl[0], sl[1]][...] * ws0
                            + u1.at[sl[0], sl[1]][...] * ws1)
            pltpu.sync_copy(u0, o_hbm.at[pl.ds(base + ch * _CW, _CW)])

        start2(0, a0, a1, semA)

        @pl.loop(0, nch, step=2)
        def _(ch):
            start2(ch + 1, c0, c1, semB)
            drain2(a0, a1, semA)
            addput(ch, a0, a1)

            @pl.when(ch + 2 < nch)
            def _():
                start2(ch + 2, a0, a1, semA)

            drain2(c0, c1, semB)
            addput(ch + 1, c0, c1)

    return k(y_sorted, p0, p1, w0, w1)


def kernel(hidden_states, gate_w, gate_proj_w, up_proj_w, down_proj_w):
    B, S, D = hidden_states.shape
    x2d = hidden_states.reshape(_T, _D)
    i1, i2, w1, w2 = _router(x2d, gate_w)
    step_block, step_expert, p0, p1 = _routing_metadata(i1, i2, w1, w2)
    x_sorted = _sc_dispatch(x2d, p0, p1)
    y_sorted = _mlp(x_sorted, gate_proj_w, up_proj_w, down_proj_w,
                    step_block, step_expert)
    w0b = jnp.broadcast_to(w1, (_T, 16))
    w1b = jnp.broadcast_to(w2, (_T, 16))
    out = _sc_combine(y_sorted, p0, p1, w0b, w1b)
    return out.reshape(B, S, D)


# speedup vs baseline: 1.3636x; 1.3636x over previous
import functools

import jax
import jax.numpy as jnp
from jax import lax
from jax.experimental import pallas as pl
from jax.experimental.pallas import tpu as pltpu
from jax.experimental.pallas import tpu_sc as plsc

_E = 8
_K = 2
_D = 2048
_INTER = 768
_T = 2048
_BT = 256
_NB = _T * _K // _BT + _E
_NPAD = _NB * _BT
_RB = 256
_CT = 16
_CW = 8


def _router_kernel(x_ref, gw_ref, i1_ref, i2_ref, w1_ref, w2_ref,
                   r1_ref, r2_ref, cnt_ref, carry):
    step = pl.program_id(0)
    x = x_ref[...]
    logits = lax.dot_general(
        x, gw_ref[...], (((1,), (1,)), ((), ())),
        preferred_element_type=jnp.float32,
        precision=lax.Precision.DEFAULT)
    lane = lax.broadcasted_iota(jnp.int32, logits.shape, 1)
    v1 = jnp.max(logits, axis=-1, keepdims=True)
    i1 = jnp.min(jnp.where(logits >= v1, lane, _E), axis=-1, keepdims=True)
    lm = jnp.where(lane == i1, -jnp.inf, logits)
    v2 = jnp.max(lm, axis=-1, keepdims=True)
    i2 = jnp.min(jnp.where(lm >= v2, lane, _E), axis=-1, keepdims=True)
    r = jnp.exp(v2 - v1)
    i1_ref[...] = i1
    i2_ref[...] = i2
    w1_ref[...] = jnp.broadcast_to(1.0 / (1.0 + r), (_RB, 16))
    w2_ref[...] = jnp.broadcast_to(r / (1.0 + r), (_RB, 16))

    oh1 = jnp.where(lane == i1, 1.0, 0.0)
    oh2 = jnp.where(lane == i2, 1.0, 0.0)
    row = lax.broadcasted_iota(jnp.int32, (_RB, _RB), 0)
    col = lax.broadcasted_iota(jnp.int32, (_RB, _RB), 1)
    L = jnp.where(row > col, 1.0, 0.0)
    exc1 = lax.dot_general(L, oh1, (((1,), (0,)), ((), ())),
                           preferred_element_type=jnp.float32,
                           precision=lax.Precision.DEFAULT)
    exc2 = lax.dot_general(L, oh2, (((1,), (0,)), ((), ())),
                           preferred_element_type=jnp.float32,
                           precision=lax.Precision.DEFAULT)

    @pl.when(step == 0)
    def _():
        carry[...] = jnp.zeros((1, _E), jnp.float32)

    cr = carry[...]
    r1m = exc1 + exc2 + cr
    r2m = r1m + oh1
    r1_ref[...] = jnp.sum(jnp.where(lane == i1, r1m, 0.0), axis=-1,
                          keepdims=True).astype(jnp.int32)
    r2_ref[...] = jnp.sum(jnp.where(lane == i2, r2m, 0.0), axis=-1,
                          keepdims=True).astype(jnp.int32)
    newc = cr + jnp.sum(oh1 + oh2, axis=0, keepdims=True)
    carry[...] = newc
    cnt_ref[...] = newc.astype(jnp.int32)


def _router(x2d, gate_w):
    return pl.pallas_call(
        _router_kernel,
        grid=(_T // _RB,),
        in_specs=[
            pl.BlockSpec((_RB, _D), lambda i: (i, 0)),
            pl.BlockSpec((_E, _D), lambda i: (0, 0)),
        ],
        out_specs=[
            pl.BlockSpec((_RB, 1), lambda i: (i, 0)),
            pl.BlockSpec((_RB, 1), lambda i: (i, 0)),
            pl.BlockSpec((_RB, 16), lambda i: (i, 0)),
            pl.BlockSpec((_RB, 16), lambda i: (i, 0)),
            pl.BlockSpec((_RB, 1), lambda i: (i, 0)),
            pl.BlockSpec((_RB, 1), lambda i: (i, 0)),
            pl.BlockSpec((1, _E), lambda i: (0, 0)),
        ],
        out_shape=[
            jax.ShapeDtypeStruct((_T, 1), jnp.int32),
            jax.ShapeDtypeStruct((_T, 1), jnp.int32),
            jax.ShapeDtypeStruct((_T, 16), jnp.float32),
            jax.ShapeDtypeStruct((_T, 16), jnp.float32),
            jax.ShapeDtypeStruct((_T, 1), jnp.int32),
            jax.ShapeDtypeStruct((_T, 1), jnp.int32),
            jax.ShapeDtypeStruct((1, _E), jnp.int32),
        ],
        scratch_shapes=[pltpu.VMEM((1, _E), jnp.float32)],
    )(x2d, gate_w)


def _meta_kernel(i1_ref, i2_ref, r1_ref, r2_ref, cnt_ref,
                 p0_ref, p1_ref, sb_ref, se_ref):
    counts = cnt_ref[...]
    padded = ((counts + _BT - 1) // _BT) * _BT
    pad_f = padded.astype(jnp.float32)
    er = lax.broadcasted_iota(jnp.int32, (_E, _E), 0)
    ec = lax.broadcasted_iota(jnp.int32, (_E, _E), 1)
    tu = jnp.where(er <= ec, 1.0, 0.0)
    bounds = lax.dot_general(pad_f, tu, (((1,), (0,)), ((), ())),
                             preferred_element_type=jnp.float32,
                             precision=lax.Precision.DEFAULT)
    pstart = bounds - pad_f
    lane_t = lax.broadcasted_iota(jnp.int32, (_T, _E), 1)
    ps_b = jnp.broadcast_to(pstart, (_T, _E))
    s0 = jnp.sum(jnp.where(lane_t == i1_ref[...], ps_b, 0.0),
                 axis=-1, keepdims=True)
    s1 = jnp.sum(jnp.where(lane_t == i2_ref[...], ps_b, 0.0),
                 axis=-1, keepdims=True)
    p0_ref[...] = (s0 + r1_ref[...].astype(jnp.float32)).astype(jnp.int32)
    p1_ref[...] = (s1 + r2_ref[...].astype(jnp.float32)).astype(jnp.int32)
    nu = jnp.sum(padded, axis=-1, keepdims=True) // _BT
    nb_i = lax.broadcasted_iota(jnp.int32, (_NB, 1), 0)
    sb = jnp.maximum(jnp.minimum(nb_i, jnp.broadcast_to(nu, (_NB, 1)) - 1),
                     0)
    sb_ref[...] = sb
    bounds_b = jnp.broadcast_to(bounds, (_NB, _E))
    v = (sb * _BT).astype(jnp.float32)
    se_ref[...] = jnp.sum(jnp.where(bounds_b <= v, 1.0, 0.0), axis=-1,
                          keepdims=True).astype(jnp.int32)


def _meta(i1, i2, r1, r2, cnt):
    return pl.pallas_call(
        _meta_kernel,
        out_shape=[
            jax.ShapeDtypeStruct((_T, 1), jnp.int32),
            jax.ShapeDtypeStruct((_T, 1), jnp.int32),
            jax.ShapeDtypeStruct((_NB, 1), jnp.int32),
            jax.ShapeDtypeStruct((_NB, 1), jnp.int32),
        ],
    )(i1, i2, r1, r2, cnt)


_NW = 32
_RPW = _NPAD // _NW
_TPW = _T // _NW


def _sc_dispatch(x2d, p0, p1):
    nch = _TPW // _CT
    i0 = p0.reshape(_NW, nch, _CT)
    i1 = p1.reshape(_NW, nch, _CT)
    mesh = plsc.VectorSubcoreMesh(core_axis_name="core",
                                  subcore_axis_name="subcore")

    @functools.partial(
        pl.kernel,
        out_type=jax.ShapeDtypeStruct((_NPAD, _D), jnp.float32),
        mesh=mesh,
        scratch_types=[
            pltpu.VMEM((nch, _CT), jnp.int32),
            pltpu.VMEM((nch, _CT), jnp.int32),
            pltpu.VMEM((_CT, _D), jnp.float32),
            pltpu.VMEM((_CT, _D), jnp.float32),
            pltpu.SemaphoreType.DMA,
            pltpu.SemaphoreType.DMA,
        ])
    def k(x_hbm, i0_hbm, i1_hbm, o_hbm, idx0_v, idx1_v, b0, b1, semR, semW):
        wid = lax.axis_index("subcore") * 2 + lax.axis_index("core")
        base = wid * _TPW
        pltpu.sync_copy(i0_hbm.at[wid], idx0_v)
        pltpu.sync_copy(i1_hbm.at[wid], idx1_v)

        def rd(c, buf):
            pltpu.async_copy(x_hbm.at[pl.ds(base + c * _CT, _CT)],
                             buf, semR)

        def drain_rd(buf):
            pltpu.make_async_copy(x_hbm.at[pl.ds(base, _CT)],
                                  buf, semR).wait()

        def scat(c, buf):
            pltpu.async_copy(buf, o_hbm.at[idx0_v.at[c]], semW)
            pltpu.async_copy(buf, o_hbm.at[idx1_v.at[c]], semW)
            pltpu.make_async_copy(buf, o_hbm.at[idx0_v.at[c]], semW).wait()
            pltpu.make_async_copy(buf, o_hbm.at[idx0_v.at[c]], semW).wait()

        rd(0, b0)

        @pl.loop(0, nch, step=2)
        def _(c):
            rd(c + 1, b1)
            drain_rd(b0)
            scat(c, b0)

            @pl.when(c + 2 < nch)
            def _():
                rd(c + 2, b0)

            drain_rd(b1)
            scat(c + 1, b1)

    return k(x2d, i0, i1)


def _mlp_kernel(sb_ref, se_ref, x_ref, g_ref, u_ref, d_ref, y_ref):
    i = pl.program_id(0)
    prev = sb_ref[jnp.maximum(i - 1, 0)]
    active = jnp.logical_or(i == 0, sb_ref[i] != prev)

    @pl.when(active)
    def _():
        x = x_ref[...]
        hg = lax.dot_general(x, g_ref[0], (((1,), (1,)), ((), ())),
                             preferred_element_type=jnp.float32,
                             precision=lax.Precision.DEFAULT)
        hu = lax.dot_general(x, u_ref[0], (((1,), (1,)), ((), ())),
                             preferred_element_type=jnp.float32,
                             precision=lax.Precision.DEFAULT)
        h = hg * jax.nn.sigmoid(hg) * hu
        y = lax.dot_general(h, d_ref[0], (((1,), (1,)), ((), ())),
                            preferred_element_type=jnp.float32,
                            precision=lax.Precision.DEFAULT)
        y_ref[...] = y


def _mlp(x_sorted, gw, uw, dw, step_block, step_expert):
    grid_spec = pltpu.PrefetchScalarGridSpec(
        num_scalar_prefetch=2,
        grid=(_NB,),
        in_specs=[
            pl.BlockSpec((_BT, _D), lambda i, sb, se: (sb[i], 0)),
            pl.BlockSpec((1, _INTER, _D), lambda i, sb, se: (se[i], 0, 0)),
            pl.BlockSpec((1, _INTER, _D), lambda i, sb, se: (se[i], 0, 0)),
            pl.BlockSpec((1, _D, _INTER), lambda i, sb, se: (se[i], 0, 0)),
        ],
        out_specs=pl.BlockSpec((_BT, _D), lambda i, sb, se: (sb[i], 0)),
    )
    return pl.pallas_call(
        _mlp_kernel,
        grid_spec=grid_spec,
        out_shape=jax.ShapeDtypeStruct((_NPAD, _D), jnp.float32),
    )(step_block, step_expert, x_sorted, gw, uw, dw)


def _sc_combine(y_sorted, p0, p1, w0, w1):
    mesh = plsc.VectorSubcoreMesh(core_axis_name="core",
                                  subcore_axis_name="subcore")

    nch = _TPW // _CW

    @functools.partial(
        pl.kernel,
        out_type=jax.ShapeDtypeStruct((_T, _D), jnp.float32),
        mesh=mesh,
        scratch_types=[
            pltpu.VMEM((_TPW,), jnp.int32),
            pltpu.VMEM((_TPW,), jnp.int32),
            pltpu.VMEM((_TPW, 16), jnp.float32),
            pltpu.VMEM((_TPW, 16), jnp.float32),
            pltpu.VMEM((_CW, _D), jnp.float32),
            pltpu.VMEM((_CW, _D), jnp.float32),
            pltpu.VMEM((_CW, _D), jnp.float32),
            pltpu.VMEM((_CW, _D), jnp.float32),
            pltpu.SemaphoreType.DMA,
            pltpu.SemaphoreType.DMA,
        ])
    def k(y_hbm, i0_hbm, i1_hbm, w0_hbm, w1_hbm, o_hbm, idx0_v, idx1_v,
          w0_v, w1_v, a0, a1, c0, c1, semA, semB):
        wid = lax.axis_index("subcore") * 2 + lax.axis_index("core")
        base = wid * _TPW
        pltpu.sync_copy(i0_hbm.at[pl.ds(base, _TPW)], idx0_v)
        pltpu.sync_copy(i1_hbm.at[pl.ds(base, _TPW)], idx1_v)
        pltpu.sync_copy(w0_hbm.at[pl.ds(base, _TPW)], w0_v)
        pltpu.sync_copy(w1_hbm.at[pl.ds(base, _TPW)], w1_v)

        def start2(ch, u0, u1, sem):
            s = ch * _CW
            pltpu.async_copy(y_hbm.at[idx0_v.at[pl.ds(s, _CW)]], u0, sem)
            pltpu.async_copy(y_hbm.at[idx1_v.at[pl.ds(s, _CW)]], u1, sem)

        def drain2(u0, u1, sem):
            pltpu.make_async_copy(y_hbm.at[idx0_v.at[pl.ds(0, _CW)]],
                                  u0, sem).wait()
            pltpu.make_async_copy(y_hbm.at[idx0_v.at[pl.ds(0, _CW)]],
                                  u1, sem).wait()

        def addput(ch, u0, u1):
            @pl.loop(0, _CW)
            def _(r):
                ws0 = w0_v[ch * _CW + r]
                ws1 = w1_v[ch * _CW + r]

                @pl.loop(0, _D, step=128)
                def _(cc):
                    for j in range(8):
                        sl = (pl.ds(r, 1), pl.ds(cc + j * 16, 16))
                        u0.at[sl[0], sl[1]][...] = (
                            u0.at[sl[0], sl[1]][...] * ws0
                            + u1.at[sl[0], sl[1]][...] * ws1)
            pltpu.sync_copy(u0, o_hbm.at[pl.ds(base + ch * _CW, _CW)])

        start2(0, a0, a1, semA)

        @pl.loop(0, nch, step=2)
        def _(ch):
            start2(ch + 1, c0, c1, semB)
            drain2(a0, a1, semA)
            addput(ch, a0, a1)

            @pl.when(ch + 2 < nch)
            def _():
                start2(ch + 2, a0, a1, semA)

            drain2(c0, c1, semB)
            addput(ch + 1, c0, c1)

    return k(y_sorted, p0, p1, w0, w1)


def kernel(hidden_states, gate_w, gate_proj_w, up_proj_w, down_proj_w):
    B, S, D = hidden_states.shape
    x2d = hidden_states.reshape(_T, _D)
    i1, i2, w1b, w2b, r1, r2, cnt = _router(x2d, gate_w)
    p0m, p1m, sbm, sem = _meta(i1, i2, r1, r2, cnt)
    p0 = p0m.reshape(-1)
    p1 = p1m.reshape(-1)
    x_sorted = _sc_dispatch(x2d, p0, p1)
    y_sorted = _mlp(x_sorted, gate_proj_w, up_proj_w, down_proj_w,
                    sbm.reshape(-1), sem.reshape(-1))
    out = _sc_combine(y_sorted, p0, p1, w1b, w2b)
    return out.reshape(B, S, D)
